# baseline (speedup 1.0000x reference)
import functools

import jax
import jax.numpy as jnp
from jax import lax
from jax.experimental import pallas as pl
from jax.experimental.pallas import tpu as pltpu
from jax.experimental.pallas import tpu_sc as plsc

NC = 2
NS = 16
L = 16
NW = NC * NS

N = 32 * 512 * 512
PER_W = N // NW
CHUNK = 32768
NCHUNK = PER_W // CHUNK
NBINS = 5


def _sc_partials(pred, targ):
    mesh = plsc.VectorSubcoreMesh(core_axis_name="c", subcore_axis_name="s")

    @functools.partial(
        pl.kernel,
        out_type=jax.ShapeDtypeStruct((NW * NBINS * L,), jnp.float32),
        mesh=mesh,
        scratch_types=[
            pltpu.VMEM((CHUNK,), jnp.float32),
            pltpu.VMEM((CHUNK,), jnp.float32),
            pltpu.VMEM((NBINS * L,), jnp.float32),
        ],
    )
    def body(pred_hbm, targ_hbm, out_hbm, pbuf, tbuf, ov):
        wid = lax.axis_index("s") * NC + lax.axis_index("c")
        base = wid * PER_W

        accs = tuple(jnp.zeros((L,), jnp.float32) for _ in range(NBINS))
        for g in range(NCHUNK):
            off = base + g * CHUNK
            pltpu.sync_copy(pred_hbm.at[pl.ds(off, CHUNK)], pbuf)
            pltpu.sync_copy(targ_hbm.at[pl.ds(off, CHUNK)], tbuf)

            def it(i, a):
                p = pbuf[pl.ds(i * L, L)]
                t = tbuf[pl.ds(i * L, L)]
                d = p - t
                d2 = d * d
                zero = jnp.zeros((L,), jnp.float32)
                return tuple(
                    a[b] + jnp.where(t >= jnp.float32(b), d2, zero)
                    for b in range(NBINS)
                )

            accs = lax.fori_loop(0, CHUNK // L, it, accs)

        for b in range(NBINS):
            ov[pl.ds(b * L, L)] = accs[b]
        pltpu.sync_copy(ov, out_hbm.at[pl.ds(wid * NBINS * L, NBINS * L)])

    return body(pred, targ)


def kernel(prediction, target, weights):
    pred = prediction.reshape(-1)
    targ = target.reshape(-1)
    partials = _sc_partials(pred, targ)
    a = partials.reshape(NW, NBINS, L).sum(axis=(0, 2))
    c = weights - jnp.concatenate([jnp.zeros((1,), weights.dtype), weights[:-1]])
    total = jnp.dot(a, c)
    return jnp.sqrt(total / N)


# speedup vs baseline: 1.5927x; 1.5927x over previous
import functools

import jax
import jax.numpy as jnp
from jax import lax
from jax.experimental import pallas as pl
from jax.experimental.pallas import tpu as pltpu
from jax.experimental.pallas import tpu_sc as plsc

NC = 2
NS = 16
L = 16
NW = NC * NS

N = 32 * 512 * 512
PER_W = N // NW
CHUNK = 16384
NCHUNK = PER_W // CHUNK
NBINS = 5
U = 8

ONE_BITS = 0x3F800000

_MESH = dict(core_axis_name="c", subcore_axis_name="s")


def _sc_fast(pred, targ):
    @functools.partial(
        pl.kernel,
        out_type=jax.ShapeDtypeStruct((NW * 2 * L,), jnp.float32),
        mesh=plsc.VectorSubcoreMesh(**_MESH),
        scratch_types=[
            pltpu.VMEM((CHUNK,), jnp.float32),
            pltpu.VMEM((CHUNK,), jnp.float32),
            pltpu.VMEM((CHUNK,), jnp.float32),
            pltpu.VMEM((CHUNK,), jnp.float32),
            pltpu.VMEM((2 * L,), jnp.float32),
            pltpu.SemaphoreType.DMA,
            pltpu.SemaphoreType.DMA,
        ],
    )
    def body(pred_hbm, targ_hbm, out_hbm, pb0, tb0, pb1, tb1, ov, sm0, sm1):
        wid = lax.axis_index("s") * NC + lax.axis_index("c")
        base = wid * PER_W
        bufs = [(pb0, tb0, sm0), (pb1, tb1, sm1)]

        def start(g):
            pb, tb, sm = bufs[g % 2]
            off = base + g * CHUNK
            hp = pltpu.async_copy(pred_hbm.at[pl.ds(off, CHUNK)], pb, sm)
            ht = pltpu.async_copy(targ_hbm.at[pl.ds(off, CHUNK)], tb, sm)
            return (hp, ht)

        zero = jnp.zeros((L,), jnp.float32)
        uzero = jnp.zeros((L,), jnp.uint32)

        def make_it(pb, tb):
            def it(i, c):
                accs = list(c[:U])
                mx0, mx1 = c[U], c[U + 1]
                b = i * (L * U)
                for j in range(U):
                    p = pb[pl.ds(b + j * L, L)]
                    t = tb[pl.ds(b + j * L, L)]
                    d = p - t
                    accs[j] = accs[j] + d * d
                    bu = lax.bitcast_convert_type(t, jnp.uint32)
                    if j % 2 == 0:
                        mx0 = jnp.where(bu > mx0, bu, mx0)
                    else:
                        mx1 = jnp.where(bu > mx1, bu, mx1)
                return (*accs, mx0, mx1)
            return it

        pending = {0: start(0)}
        carry = tuple(zero for _ in range(U)) + (uzero, uzero)
        for g in range(NCHUNK):
            if g + 1 < NCHUNK:
                pending[g + 1] = start(g + 1)
            for h in pending.pop(g):
                h.wait()
            pb, tb, _ = bufs[g % 2]
            carry = lax.fori_loop(0, CHUNK // (L * U), make_it(pb, tb), carry)

        acc = carry[0]
        for j in range(1, U):
            acc = acc + carry[j]
        mx0, mx1 = carry[U], carry[U + 1]
        mx = jnp.where(mx0 > mx1, mx0, mx1)
        ov[pl.ds(0, L)] = acc
        ov[pl.ds(L, L)] = lax.bitcast_convert_type(mx, jnp.float32)
        pltpu.sync_copy(ov, out_hbm.at[pl.ds(wid * 2 * L, 2 * L)])

    return body(pred, targ)


def _sc_general(pred, targ):
    @functools.partial(
        pl.kernel,
        out_type=jax.ShapeDtypeStruct((NW * NBINS * L,), jnp.float32),
        mesh=plsc.VectorSubcoreMesh(**_MESH),
        scratch_types=[
            pltpu.VMEM((CHUNK,), jnp.float32),
            pltpu.VMEM((CHUNK,), jnp.float32),
            pltpu.VMEM((NBINS * L,), jnp.float32),
        ],
    )
    def body(pred_hbm, targ_hbm, out_hbm, pbuf, tbuf, ov):
        wid = lax.axis_index("s") * NC + lax.axis_index("c")
        base = wid * PER_W

        zero = jnp.zeros((L,), jnp.float32)
        accs = tuple(zero for _ in range(NBINS))
        for g in range(NCHUNK):
            off = base + g * CHUNK
            pltpu.sync_copy(pred_hbm.at[pl.ds(off, CHUNK)], pbuf)
            pltpu.sync_copy(targ_hbm.at[pl.ds(off, CHUNK)], tbuf)

            def it(i, a):
                p = pbuf[pl.ds(i * L, L)]
                t = tbuf[pl.ds(i * L, L)]
                d = p - t
                d2 = d * d
                return tuple(
                    a[b] + jnp.where(t >= jnp.float32(b), d2, zero)
                    for b in range(NBINS)
                )

            accs = lax.fori_loop(0, CHUNK // L, it, accs)

        for b in range(NBINS):
            ov[pl.ds(b * L, L)] = accs[b]
        pltpu.sync_copy(ov, out_hbm.at[pl.ds(wid * NBINS * L, NBINS * L)])

    return body(pred, targ)


def kernel(prediction, target, weights):
    pred = prediction.reshape(-1)
    targ = target.reshape(-1)

    fast = _sc_fast(pred, targ).reshape(NW, 2, L)
    s_fast = fast[:, 0, :].sum()
    mx = lax.bitcast_convert_type(fast[:, 1, :], jnp.uint32).max()
    clean = mx < jnp.uint32(ONE_BITS)

    def fast_fn(_):
        return jnp.sqrt(s_fast * weights[0] / N)

    def general_fn(_):
        partials = _sc_general(pred, targ)
        a = partials.reshape(NW, NBINS, L).sum(axis=(0, 2))
        c = weights - jnp.concatenate(
            [jnp.zeros((1,), weights.dtype), weights[:-1]])
        return jnp.sqrt(jnp.dot(a, c) / N)

    return lax.cond(clean, fast_fn, general_fn, None)


# speedup vs baseline: 3.1853x; 1.9999x over previous
import functools

import jax
import jax.numpy as jnp
from jax import lax
from jax.experimental import pallas as pl
from jax.experimental.pallas import tpu as pltpu
from jax.experimental.pallas import tpu_sc as plsc

NC = 2
NS = 16
L = 16
NW = NC * NS

B = 32
H = 512
W = 512
N = B * H * W
R = 32
NCHUNK = H // R
NBINS = 5
U = 8

ONE_BITS = 0x3F800000

_MESH = dict(core_axis_name="c", subcore_axis_name="s")


def _sc_fast(pred, targ):
    @functools.partial(
        pl.kernel,
        out_type=jax.ShapeDtypeStruct((NW * 2 * L,), jnp.float32),
        mesh=plsc.VectorSubcoreMesh(**_MESH),
        scratch_types=[
            pltpu.VMEM((R, W), jnp.float32),
            pltpu.VMEM((R, W), jnp.float32),
            pltpu.VMEM((R, W), jnp.float32),
            pltpu.VMEM((R, W), jnp.float32),
            pltpu.VMEM((2 * L,), jnp.float32),
            pltpu.SemaphoreType.DMA,
            pltpu.SemaphoreType.DMA,
        ],
    )
    def body(pred_hbm, targ_hbm, out_hbm, pb0, tb0, pb1, tb1, ov, sm0, sm1):
        wid = lax.axis_index("s") * NC + lax.axis_index("c")
        bufs = [(pb0, tb0, sm0), (pb1, tb1, sm1)]

        def start(g):
            pb, tb, sm = bufs[g % 2]
            r0 = g * R
            hp = pltpu.async_copy(pred_hbm.at[wid, pl.ds(r0, R), :], pb, sm)
            ht = pltpu.async_copy(targ_hbm.at[wid, pl.ds(r0, R), :], tb, sm)
            return (hp, ht)

        zero = jnp.zeros((L,), jnp.float32)
        uzero = jnp.zeros((L,), jnp.uint32)
        blocks_per_row = W // (L * U)

        def make_it(pb, tb):
            def it(i, c):
                accs = list(c[:U])
                mx0, mx1 = c[U], c[U + 1]
                row = lax.shift_right_logical(i, 2)
                cb = pl.multiple_of(
                    lax.shift_left(lax.bitwise_and(i, 3), 7), 128)
                for j in range(U):
                    p = pb[row, pl.ds(cb + j * L, L)]
                    t = tb[row, pl.ds(cb + j * L, L)]
                    d = p - t
                    accs[j] = accs[j] + d * d
                    bu = lax.bitcast_convert_type(t, jnp.uint32)
                    if j % 2 == 0:
                        mx0 = jnp.where(bu > mx0, bu, mx0)
                    else:
                        mx1 = jnp.where(bu > mx1, bu, mx1)
                return (*accs, mx0, mx1)
            return it

        pending = {0: start(0)}
        carry = tuple(zero for _ in range(U)) + (uzero, uzero)
        for g in range(NCHUNK):
            if g + 1 < NCHUNK:
                pending[g + 1] = start(g + 1)
            for h in pending.pop(g):
                h.wait()
            pb, tb, _ = bufs[g % 2]
            carry = lax.fori_loop(0, R * blocks_per_row, make_it(pb, tb),
                                  carry)

        acc = carry[0]
        for j in range(1, U):
            acc = acc + carry[j]
        mx0, mx1 = carry[U], carry[U + 1]
        mx = jnp.where(mx0 > mx1, mx0, mx1)
        ov[pl.ds(0, L)] = acc
        ov[pl.ds(L, L)] = lax.bitcast_convert_type(mx, jnp.float32)
        pltpu.sync_copy(ov, out_hbm.at[pl.ds(wid * 2 * L, 2 * L)])

    return body(pred, targ)


def _sc_general(pred, targ):
    @functools.partial(
        pl.kernel,
        out_type=jax.ShapeDtypeStruct((NW * NBINS * L,), jnp.float32),
        mesh=plsc.VectorSubcoreMesh(**_MESH),
        scratch_types=[
            pltpu.VMEM((R, W), jnp.float32),
            pltpu.VMEM((R, W), jnp.float32),
            pltpu.VMEM((NBINS * L,), jnp.float32),
        ],
    )
    def body(pred_hbm, targ_hbm, out_hbm, pbuf, tbuf, ov):
        wid = lax.axis_index("s") * NC + lax.axis_index("c")

        zero = jnp.zeros((L,), jnp.float32)
        accs = tuple(zero for _ in range(NBINS))
        for g in range(NCHUNK):
            r0 = g * R
            pltpu.sync_copy(pred_hbm.at[wid, pl.ds(r0, R), :], pbuf)
            pltpu.sync_copy(targ_hbm.at[wid, pl.ds(r0, R), :], tbuf)

            def it(i, a):
                row = lax.shift_right_logical(i, 5)
                col = pl.multiple_of(
                    lax.shift_left(lax.bitwise_and(i, 31), 4), 16)
                p = pbuf[row, pl.ds(col, L)]
                t = tbuf[row, pl.ds(col, L)]
                d = p - t
                d2 = d * d
                return tuple(
                    a[b] + jnp.where(t >= jnp.float32(b), d2, zero)
                    for b in range(NBINS)
                )

            accs = lax.fori_loop(0, R * (W // L), it, accs)

        for b in range(NBINS):
            ov[pl.ds(b * L, L)] = accs[b]
        pltpu.sync_copy(ov, out_hbm.at[pl.ds(wid * NBINS * L, NBINS * L)])

    return body(pred, targ)


def kernel(prediction, target, weights):
    fast = _sc_fast(prediction, target).reshape(NW, 2, L)
    s_fast = fast[:, 0, :].sum()
    mx = lax.bitcast_convert_type(fast[:, 1, :], jnp.uint32).max()
    clean = mx < jnp.uint32(ONE_BITS)

    def fast_fn(_):
        return jnp.sqrt(s_fast * weights[0] / N)

    def general_fn(_):
        partials = _sc_general(prediction, target)
        a = partials.reshape(NW, NBINS, L).sum(axis=(0, 2))
        c = weights - jnp.concatenate(
            [jnp.zeros((1,), weights.dtype), weights[:-1]])
        return jnp.sqrt(jnp.dot(a, c) / N)

    return lax.cond(clean, fast_fn, general_fn, None)
